# baseline (speedup 1.0000x reference)
import functools

import jax
import jax.numpy as jnp
from jax import lax
from jax.experimental import pallas as pl
from jax.experimental.pallas import tpu as pltpu
from jax.experimental.pallas import tpu_sc as plsc

_CB = 1024
_D = 64
_BLK = 512


def _vq_tc_body(zsq_ref, wsq_ref, z_ref, w_ref, idx_ref, loss_ref):
    zb = z_ref[...]
    w = w_ref[...]
    m = lax.dot_general(zb, w, (((1,), (1,)), ((), ())),
                        preferred_element_type=jnp.float32)
    dist = (zsq_ref[...] + wsq_ref[...]) - 2.0 * m
    minval = jnp.min(dist, axis=1, keepdims=True)
    cols = lax.broadcasted_iota(jnp.int32, (_BLK, _CB), 1)
    idx = jnp.min(jnp.where(dist == minval, cols, _CB), axis=1)
    idx_ref[...] = idx.reshape(1, 1, _BLK)
    loss_ref[0, 0, 0] = jnp.sum(minval)


def _tc_stage(flat_z, W, zsq, wsq, interpret=False):
    n = flat_z.shape[0]
    nblk = n // _BLK
    return pl.pallas_call(
        _vq_tc_body,
        grid=(nblk,),
        in_specs=[
            pl.BlockSpec((_BLK, 1), lambda i: (i, 0)),
            pl.BlockSpec((1, _CB), lambda i: (0, 0)),
            pl.BlockSpec((_BLK, _D), lambda i: (i, 0)),
            pl.BlockSpec((_CB, _D), lambda i: (0, 0)),
        ],
        out_specs=[
            pl.BlockSpec((1, 1, _BLK), lambda i: (i, 0, 0)),
            pl.BlockSpec((1, 1, 1), lambda i: (i, 0, 0), memory_space=pltpu.SMEM),
        ],
        out_shape=[
            jax.ShapeDtypeStruct((nblk, 1, _BLK), jnp.int32),
            jax.ShapeDtypeStruct((nblk, 1, 1), jnp.float32),
        ],
        interpret=interpret,
    )(zsq, wsq, flat_z, W)


def kernel(z, W):
    flat_z = z.reshape(-1, W.shape[1])
    n = flat_z.shape[0]
    zsq = jnp.sum(flat_z ** 2, axis=1, keepdims=True)
    wsq = jnp.sum(W ** 2, axis=1).reshape(1, -1)
    idx3, loss_parts = _tc_stage(flat_z, W, zsq, wsq)
    idx_flat = idx3.reshape(-1)
    quantized = jnp.take(W, idx_flat, axis=0)
    quantized_st = (flat_z + (quantized - flat_z)).reshape(z.shape)
    vq_loss = 2.0 * (jnp.sum(loss_parts) / (n * W.shape[1]))
    return quantized_st, idx_flat.reshape(z.shape[:-1]), vq_loss


# speedup vs baseline: 1.3370x; 1.3370x over previous
import functools

import jax
import jax.numpy as jnp
from jax import lax
from jax.experimental import pallas as pl
from jax.experimental.pallas import tpu as pltpu
from jax.experimental.pallas import tpu_sc as plsc

_CB = 1024
_D = 64
_BLK = 512


def _vq_tc_body(zsq_ref, wsq_ref, z_ref, w_ref, idx_ref, loss_ref):
    zb = z_ref[...]
    w = w_ref[...]
    m = lax.dot_general(zb, w, (((1,), (1,)), ((), ())),
                        preferred_element_type=jnp.float32)
    dist = (zsq_ref[...] + wsq_ref[...]) - 2.0 * m
    minval = jnp.min(dist, axis=1, keepdims=True)
    cols = lax.broadcasted_iota(jnp.int32, (_BLK, _CB), 1)
    idx = jnp.min(jnp.where(dist == minval, cols, _CB), axis=1)
    idx_ref[...] = idx.reshape(1, 1, _BLK)
    loss_ref[0, 0, 0] = jnp.sum(minval)


def _tc_stage(flat_z, W, zsq, wsq, interpret=False):
    n = flat_z.shape[0]
    nblk = n // _BLK
    return pl.pallas_call(
        _vq_tc_body,
        grid=(nblk,),
        in_specs=[
            pl.BlockSpec((_BLK, 1), lambda i: (i, 0)),
            pl.BlockSpec((1, _CB), lambda i: (0, 0)),
            pl.BlockSpec((_BLK, _D), lambda i: (i, 0)),
            pl.BlockSpec((_CB, _D), lambda i: (0, 0)),
        ],
        out_specs=[
            pl.BlockSpec((1, 1, _BLK), lambda i: (i, 0, 0)),
            pl.BlockSpec((1, 1, 1), lambda i: (i, 0, 0), memory_space=pltpu.SMEM),
        ],
        out_shape=[
            jax.ShapeDtypeStruct((nblk, 1, _BLK), jnp.int32),
            jax.ShapeDtypeStruct((nblk, 1, 1), jnp.float32),
        ],
        interpret=interpret,
    )(zsq, wsq, flat_z, W)


_NC = 2
_NS = 16
_NW = _NC * _NS
_CHUNK = 128


def _sc_gather(W, idx2d, n):
    bpw = n // _NW
    cpw = bpw // _CHUNK
    mesh = plsc.VectorSubcoreMesh(core_axis_name="c", subcore_axis_name="s")

    @functools.partial(
        pl.kernel,
        mesh=mesh,
        out_type=jax.ShapeDtypeStruct((n, _D), jnp.float32),
        scratch_types=[
            pltpu.VMEM((cpw, _CHUNK), jnp.int32),
            pltpu.VMEM((bpw, _D), jnp.float32),
            pltpu.SemaphoreType.DMA,
        ],
        compiler_params=pltpu.CompilerParams(use_tc_tiling_on_sc=False),
    )
    def gk(w_hbm, idx_hbm, out_hbm, idx_v, rows_v, sem):
        wid = lax.axis_index("s") * _NC + lax.axis_index("c")
        pltpu.sync_copy(idx_hbm.at[pl.ds(wid * cpw, cpw)], idx_v)
        copies = [
            pltpu.async_copy(
                w_hbm.at[idx_v.at[j]],
                rows_v.at[pl.ds(j * _CHUNK, _CHUNK)],
                sem,
            )
            for j in range(cpw)
        ]
        for c in copies:
            c.wait()
        pltpu.sync_copy(rows_v, out_hbm.at[pl.ds(wid * bpw, bpw)])

    return gk(W, idx2d)


def kernel(z, W):
    flat_z = z.reshape(-1, W.shape[1])
    n = flat_z.shape[0]
    zsq = jnp.sum(flat_z ** 2, axis=1, keepdims=True)
    wsq = jnp.sum(W ** 2, axis=1).reshape(1, -1)
    idx3, loss_parts = _tc_stage(flat_z, W, zsq, wsq)
    idx_flat = idx3.reshape(-1)
    quantized = _sc_gather(W, idx_flat.reshape(-1, _CHUNK), n)
    quantized_st = quantized.reshape(z.shape)
    vq_loss = 2.0 * (jnp.sum(loss_parts) / (n * W.shape[1]))
    return quantized_st, idx_flat.reshape(z.shape[:-1]), vq_loss


# speedup vs baseline: 1.3720x; 1.0262x over previous
import functools

import jax
import jax.numpy as jnp
from jax import lax
from jax.experimental import pallas as pl
from jax.experimental.pallas import tpu as pltpu
from jax.experimental.pallas import tpu_sc as plsc

_CB = 1024
_D = 64
_BLK = 1024


def _vq_tc_body(wsq_ref, z_ref, w2_ref, idx_ref, loss_ref):
    zb = z_ref[...]
    w2 = w2_ref[...]
    zsq = jnp.sum(zb ** 2, axis=1, keepdims=True)
    m2 = lax.dot_general(zb, w2, (((1,), (1,)), ((), ())),
                         preferred_element_type=jnp.float32)
    dist = (zsq + wsq_ref[...]) - m2
    minval = jnp.min(dist, axis=1, keepdims=True)
    cols = lax.broadcasted_iota(jnp.int32, (_BLK, _CB), 1)
    idx = jnp.min(jnp.where(dist == minval, cols, _CB), axis=1)
    idx_ref[...] = idx.reshape(1, 1, _BLK)
    loss_ref[0, 0, 0] = jnp.sum(minval)


def _tc_stage(flat_z, W2, wsq, interpret=False):
    n = flat_z.shape[0]
    nblk = n // _BLK
    return pl.pallas_call(
        _vq_tc_body,
        grid=(nblk,),
        in_specs=[
            pl.BlockSpec((1, _CB), lambda i: (0, 0)),
            pl.BlockSpec((_BLK, _D), lambda i: (i, 0)),
            pl.BlockSpec((_CB, _D), lambda i: (0, 0)),
        ],
        out_specs=[
            pl.BlockSpec((1, 1, _BLK), lambda i: (i, 0, 0)),
            pl.BlockSpec((1, 1, 1), lambda i: (i, 0, 0), memory_space=pltpu.SMEM),
        ],
        out_shape=[
            jax.ShapeDtypeStruct((nblk, 1, _BLK), jnp.int32),
            jax.ShapeDtypeStruct((nblk, 1, 1), jnp.float32),
        ],
        interpret=interpret,
    )(wsq, flat_z, W2)


_NC = 2
_NS = 16
_NW = _NC * _NS
_CHUNK = 128


def _sc_gather(W, idx3, out_shape):
    B, T = out_shape[0], out_shape[1]
    n = B * T
    bpw = n // _NW
    cpw = bpw // _CHUNK
    wpb = T // bpw
    mesh = plsc.VectorSubcoreMesh(core_axis_name="c", subcore_axis_name="s")

    @functools.partial(
        pl.kernel,
        mesh=mesh,
        out_type=jax.ShapeDtypeStruct((B, T, _D), jnp.float32),
        scratch_types=[
            pltpu.VMEM((bpw,), jnp.int32),
            pltpu.VMEM((bpw, _D), jnp.float32),
            pltpu.SemaphoreType.DMA,
        ],
        compiler_params=pltpu.CompilerParams(use_tc_tiling_on_sc=False),
    )
    def gk(w_hbm, idx_hbm, out_hbm, idx_v, rows_v, sem):
        wid = lax.axis_index("s") * _NC + lax.axis_index("c")
        b = wid // wpb
        off = (wid % wpb) * bpw
        pltpu.sync_copy(idx_hbm.at[b, 0, pl.ds(off, bpw)], idx_v)
        copies = [
            pltpu.async_copy(
                w_hbm.at[idx_v.at[pl.ds(j * _CHUNK, _CHUNK)]],
                rows_v.at[pl.ds(j * _CHUNK, _CHUNK)],
                sem,
            )
            for j in range(cpw)
        ]
        for c in copies:
            c.wait()
        pltpu.sync_copy(rows_v, out_hbm.at[b, pl.ds(off, bpw)])

    return gk(W, idx3)


def kernel(z, W):
    flat_z = z.reshape(-1, W.shape[1])
    n = flat_z.shape[0]
    wsq = jnp.sum(W ** 2, axis=1).reshape(1, -1)
    idx3, loss_parts = _tc_stage(flat_z, W + W, wsq)
    quantized_st = _sc_gather(W, idx3, z.shape)
    vq_loss = 2.0 * (jnp.sum(loss_parts) / (n * W.shape[1]))
    return quantized_st, idx3.reshape(z.shape[:-1]), vq_loss


# speedup vs baseline: 1.4844x; 1.0819x over previous
import functools

import jax
import jax.numpy as jnp
from jax import lax
from jax.experimental import pallas as pl
from jax.experimental.pallas import tpu as pltpu
from jax.experimental.pallas import tpu_sc as plsc

_CB = 1024
_D = 64
_BLK = 1024


_RT = 128
_G = _CB // 128


def _vq_tc_body(wsq_ref, z_ref, w2_ref, idx_ref, loss_ref):
    zb = z_ref[...]
    w2 = w2_ref[...]
    zsq = jnp.sum(zb ** 2, axis=1, keepdims=True)
    m2 = lax.dot_general(zb, w2, (((1,), (1,)), ((), ())),
                         preferred_element_type=jnp.float32)
    wsq = wsq_ref[...]
    lanes = lax.broadcasted_iota(jnp.int32, (_RT, 128), 1)
    idx_parts = []
    loss = jnp.float32(0.0)
    for r in range(_BLK // _RT):
        zsq_r = zsq[r * _RT:(r + 1) * _RT, :]
        run_min = (zsq_r + wsq[:, 0:128]) - m2[r * _RT:(r + 1) * _RT, 0:128]
        run_g = jnp.zeros((_RT, 128), jnp.int32)
        for g in range(1, _G):
            d_g = (zsq_r + wsq[:, g * 128:(g + 1) * 128]) \
                - m2[r * _RT:(r + 1) * _RT, g * 128:(g + 1) * 128]
            better = d_g < run_min
            run_min = jnp.minimum(d_g, run_min)
            run_g = jnp.where(better, jnp.int32(g), run_g)
        minval = jnp.min(run_min, axis=1, keepdims=True)
        key = jnp.where(run_min == minval, run_g * 128 + lanes, _CB)
        idx_parts.append(jnp.min(key, axis=1))
        loss = loss + jnp.sum(minval)
    idx = jnp.concatenate(idx_parts, axis=0)
    idx_ref[...] = idx.reshape(1, 1, _BLK)
    loss_ref[0, 0, 0] = loss


def _tc_stage(flat_z, W2, wsq, interpret=False):
    n = flat_z.shape[0]
    nblk = n // _BLK
    return pl.pallas_call(
        _vq_tc_body,
        grid=(nblk,),
        in_specs=[
            pl.BlockSpec((1, _CB), lambda i: (0, 0)),
            pl.BlockSpec((_BLK, _D), lambda i: (i, 0)),
            pl.BlockSpec((_CB, _D), lambda i: (0, 0)),
        ],
        out_specs=[
            pl.BlockSpec((1, 1, _BLK), lambda i: (i, 0, 0)),
            pl.BlockSpec((1, 1, 1), lambda i: (i, 0, 0), memory_space=pltpu.SMEM),
        ],
        out_shape=[
            jax.ShapeDtypeStruct((nblk, 1, _BLK), jnp.int32),
            jax.ShapeDtypeStruct((nblk, 1, 1), jnp.float32),
        ],
        interpret=interpret,
    )(wsq, flat_z, W2)


_NC = 2
_NS = 16
_NW = _NC * _NS
_CHUNK = 128


def _sc_gather(W, idx3, out_shape):
    B, T = out_shape[0], out_shape[1]
    n = B * T
    bpw = n // _NW
    cpw = bpw // _CHUNK
    wpb = T // bpw
    mesh = plsc.VectorSubcoreMesh(core_axis_name="c", subcore_axis_name="s")

    @functools.partial(
        pl.kernel,
        mesh=mesh,
        out_type=jax.ShapeDtypeStruct((B, T, _D), jnp.float32),
        scratch_types=[
            pltpu.VMEM((bpw,), jnp.int32),
            pltpu.VMEM((bpw, _D), jnp.float32),
            pltpu.SemaphoreType.DMA,
        ],
        compiler_params=pltpu.CompilerParams(use_tc_tiling_on_sc=False),
    )
    def gk(w_hbm, idx_hbm, out_hbm, idx_v, rows_v, sem):
        wid = lax.axis_index("s") * _NC + lax.axis_index("c")
        b = wid // wpb
        off = (wid % wpb) * bpw
        pltpu.sync_copy(idx_hbm.at[b, 0, pl.ds(off, bpw)], idx_v)
        copies = [
            pltpu.async_copy(
                w_hbm.at[idx_v.at[pl.ds(j * _CHUNK, _CHUNK)]],
                rows_v.at[pl.ds(j * _CHUNK, _CHUNK)],
                sem,
            )
            for j in range(cpw)
        ]
        for c in copies:
            c.wait()
        pltpu.sync_copy(rows_v, out_hbm.at[b, pl.ds(off, bpw)])

    return gk(W, idx3)


def kernel(z, W):
    flat_z = z.reshape(-1, W.shape[1])
    n = flat_z.shape[0]
    wsq = jnp.sum(W ** 2, axis=1).reshape(1, -1)
    idx3, loss_parts = _tc_stage(flat_z, W + W, wsq)
    quantized_st = _sc_gather(W, idx3, z.shape)
    vq_loss = 2.0 * (jnp.sum(loss_parts) / (n * W.shape[1]))
    return quantized_st, idx3.reshape(z.shape[:-1]), vq_loss


# speedup vs baseline: 1.8930x; 1.2753x over previous
import functools

import jax
import jax.numpy as jnp
from jax import lax
from jax.experimental import pallas as pl
from jax.experimental.pallas import tpu as pltpu
from jax.experimental.pallas import tpu_sc as plsc

_CB = 1024
_D = 64
_BLK = 8192
_SB = 1024
_RT = 128
_G = _CB // 128


def _vq_tc_body(wsq_ref, z_ref, w2_ref, idxl_ref, idxs_ref, loss_ref):
    w2 = w2_ref[...]
    wsq = wsq_ref[...]
    lanes = lax.broadcasted_iota(jnp.int32, (_RT, 128), 1)
    loss = jnp.float32(0.0)
    for sb in range(_BLK // _SB):
        zb = z_ref[sb * _SB:(sb + 1) * _SB, :]
        zsq = jnp.sum(zb ** 2, axis=1, keepdims=True)
        m2 = lax.dot_general(zb, w2, (((1,), (1,)), ((), ())),
                             preferred_element_type=jnp.float32)
        for r in range(_SB // _RT):
            zsq_r = zsq[r * _RT:(r + 1) * _RT, :]
            run_min = (zsq_r + wsq[:, 0:128]) \
                - m2[r * _RT:(r + 1) * _RT, 0:128]
            run_g = jnp.zeros((_RT, 128), jnp.int32)
            for g in range(1, _G):
                d_g = (zsq_r + wsq[:, g * 128:(g + 1) * 128]) \
                    - m2[r * _RT:(r + 1) * _RT, g * 128:(g + 1) * 128]
                better = d_g < run_min
                run_min = jnp.minimum(d_g, run_min)
                run_g = jnp.where(better, jnp.int32(g), run_g)
            minval = jnp.min(run_min, axis=1, keepdims=True)
            key = jnp.where(run_min == minval, run_g * 128 + lanes, _CB)
            part = jnp.min(key, axis=1)
            idxl_ref[sb, r * _RT:(r + 1) * _RT] = part
            idxs_ref[sb * (_SB // _RT) + r, :] = part
            loss = loss + jnp.sum(minval)
    loss_ref[0, 0, 0] = loss


def _tc_stage(flat_z, W2, wsq, interpret=False):
    n = flat_z.shape[0]
    nblk = n // _BLK
    rows = n // _RT
    return pl.pallas_call(
        _vq_tc_body,
        grid=(nblk,),
        in_specs=[
            pl.BlockSpec((1, _CB), lambda i: (0, 0)),
            pl.BlockSpec((_BLK, _D), lambda i: (i, 0)),
            pl.BlockSpec((_CB, _D), lambda i: (0, 0)),
        ],
        out_specs=[
            pl.BlockSpec((_BLK // _SB, _SB), lambda i: (i, 0)),
            pl.BlockSpec((_BLK // _RT, 128), lambda i: (i, 0)),
            pl.BlockSpec((1, 1, 1), lambda i: (i, 0, 0), memory_space=pltpu.SMEM),
        ],
        out_shape=[
            jax.ShapeDtypeStruct((n // _SB, _SB), jnp.int32),
            jax.ShapeDtypeStruct((rows, 128), jnp.int32),
            jax.ShapeDtypeStruct((nblk, 1, 1), jnp.float32),
        ],
        interpret=interpret,
    )(wsq, flat_z, W2)


_NC = 2
_NS = 16
_NW = _NC * _NS
_CHUNK = 128


def _sc_gather(W, idx2d, out_shape):
    B, T = out_shape[0], out_shape[1]
    n = B * T
    bpw = n // _NW
    cpw = bpw // _CHUNK
    wpb = T // bpw
    mesh = plsc.VectorSubcoreMesh(core_axis_name="c", subcore_axis_name="s")

    @functools.partial(
        pl.kernel,
        mesh=mesh,
        out_type=jax.ShapeDtypeStruct((B, T, _D), jnp.float32),
        scratch_types=[
            pltpu.VMEM((cpw, _CHUNK), jnp.int32),
            pltpu.VMEM((bpw, _D), jnp.float32),
            pltpu.SemaphoreType.DMA,
        ],
        compiler_params=pltpu.CompilerParams(use_tc_tiling_on_sc=False),
    )
    def gk(w_hbm, idx_hbm, out_hbm, idx_v, rows_v, sem):
        wid = lax.axis_index("s") * _NC + lax.axis_index("c")
        b = wid // wpb
        off = (wid % wpb) * bpw
        pltpu.sync_copy(idx_hbm.at[pl.ds(wid * cpw, cpw)], idx_v)
        copies = [
            pltpu.async_copy(
                w_hbm.at[idx_v.at[j]],
                rows_v.at[pl.ds(j * _CHUNK, _CHUNK)],
                sem,
            )
            for j in range(cpw)
        ]
        for c in copies:
            c.wait()
        pltpu.sync_copy(rows_v, out_hbm.at[b, pl.ds(off, bpw)])

    return gk(W, idx2d)


def kernel(z, W):
    flat_z = z.reshape(-1, W.shape[1])
    n = flat_z.shape[0]
    wsq = jnp.sum(W ** 2, axis=1).reshape(1, -1)
    idxl, idxs, loss_parts = _tc_stage(flat_z, W + W, wsq)
    quantized_st = _sc_gather(W, idxs, z.shape)
    vq_loss = 2.0 * (jnp.sum(loss_parts) / (n * W.shape[1]))
    return quantized_st, idxl, vq_loss
